# Initial kernel scaffold; baseline (speedup 1.0000x reference)
import jax
import jax.numpy as jnp
from jax.experimental import pallas as pl


def kernel(confidences, predictions, labels):
    raise NotImplementedError("write your pallas kernel here")


# speedup vs baseline: 2.0060x; 2.0060x over previous
import jax
import jax.numpy as jnp
import numpy as np
from jax import lax
from jax.experimental import pallas as pl
from jax.experimental.pallas import tpu as pltpu
from jax.experimental.pallas import tpu_sc as plsc

_N_BINS = 15
_L = 16
_UNROLL = 8

_BOUNDS = np.linspace(0.0, 1.0, _N_BINS + 1).astype(np.float32)


def _ece_partials(conf, pred, lab, *, num_cores, num_subcores, elems_per_worker):
    nw = num_cores * num_subcores
    nv = elems_per_worker // _L
    assert nv % _UNROLL == 0

    def body(conf_hbm, pred_hbm, lab_hbm, out_hbm,
             conf_v, pred_v, lab_v, acc_v, buf_v, sem):
        wid = lax.axis_index("s") * num_cores + lax.axis_index("c")
        base = wid * elems_per_worker
        cp_c = pltpu.async_copy(conf_hbm.at[pl.ds(base, elems_per_worker)], conf_v, sem)
        cp_p = pltpu.async_copy(pred_hbm.at[pl.ds(base, elems_per_worker)], pred_v, sem)
        cp_l = pltpu.async_copy(lab_hbm.at[pl.ds(base, elems_per_worker)], lab_v, sem)

        zero = jnp.zeros((_L,), jnp.float32)
        for r in range(_L):
            acc_v[r, :] = zero
        lane = lax.iota(jnp.int32, _L)
        tabv = lane.astype(jnp.float32) / jnp.float32(_N_BINS)

        cp_c.wait()
        cp_p.wait()
        cp_l.wait()

        def one(off):
            c = conf_v[pl.ds(off, _L)]
            p = pred_v[pl.ds(off, _L)]
            l = lab_v[pl.ds(off, _L)]
            a = jnp.where(p == l, jnp.float32(1.0), jnp.float32(0.0))
            d = c - a
            ti = (c * jnp.float32(15.0)).astype(jnp.int32)
            bi = lax.bitcast_convert_type(
                jnp.minimum(lax.bitcast_convert_type(ti, jnp.uint32),
                            jnp.uint32(14)),
                jnp.int32)
            lo = jnp.take_along_axis(tabv, bi, axis=0)
            b = bi - (c == lo).astype(jnp.int32)
            plsc.addupdate_scatter(acc_v, [lane, b], d,
                                   mask=c > jnp.float32(0.0))

        @plsc.parallel_loop(0, nv * _L, _L, unroll=_UNROLL)
        def _(off):
            one(off)

        tot = acc_v[0, :]
        for r in range(1, _L):
            tot = tot + acc_v[r, :]
        buf_v[...] = tot
        pltpu.sync_copy(buf_v, out_hbm.at[wid])

    mesh = plsc.VectorSubcoreMesh(
        core_axis_name="c", subcore_axis_name="s",
        num_cores=num_cores, num_subcores=num_subcores)
    kfn = pl.kernel(
        body,
        out_type=jax.ShapeDtypeStruct((nw, _L), jnp.float32),
        mesh=mesh,
        compiler_params=pltpu.CompilerParams(needs_layout_passes=False),
        scratch_types=[
            pltpu.VMEM((elems_per_worker,), jnp.float32),
            pltpu.VMEM((elems_per_worker,), jnp.int32),
            pltpu.VMEM((elems_per_worker,), jnp.int32),
            pltpu.VMEM((_L, _L), jnp.float32),
            pltpu.VMEM((_L,), jnp.float32),
            pltpu.SemaphoreType.DMA,
        ],
    )
    return kfn(conf, pred, lab)


@jax.jit
def kernel(confidences, predictions, labels):
    num_cores, num_subcores = 2, 16
    nw = num_cores * num_subcores
    n = confidences.shape[0]
    chunk = _L * _UNROLL * nw
    padded = ((n + chunk - 1) // chunk) * chunk
    epw = padded // nw

    pad = padded - n
    conf = jnp.pad(confidences, (0, pad))
    pred = jnp.pad(predictions, (0, pad))
    lab = jnp.pad(labels, (0, pad))

    parts = _ece_partials(conf, pred, lab,
                          num_cores=num_cores, num_subcores=num_subcores,
                          elems_per_worker=epw)
    sums = parts.sum(axis=0)
    ece = jnp.abs(sums[:_N_BINS]).sum() / jnp.float32(n)
    return ece.reshape(1)


# speedup vs baseline: 2.4169x; 1.2048x over previous
import jax
import jax.numpy as jnp
from jax import lax
from jax.experimental import pallas as pl
from jax.experimental.pallas import tpu as pltpu
from jax.experimental.pallas import tpu_sc as plsc

_N_BINS = 15
_L = 16
_UNROLL = 8


def _ece_partials(conf, pred, lab, *, num_cores, num_subcores):
    nw = num_cores * num_subcores
    n = conf.shape[0]
    assert n % _L == 0
    total_vec = n // _L
    base_vec = total_vec // nw
    nbig = total_vec - base_vec * nw
    nv = base_vec + (1 if nbig else 0)
    short_elems = base_vec * _L
    epw = nv * _L
    nv_main = (nv // _UNROLL) * _UNROLL

    def body(conf_hbm, pred_hbm, lab_hbm, out_hbm,
             conf_v, pred_v, lab_v, acc_v, buf_v, sem):
        wid = lax.axis_index("s") * num_cores + lax.axis_index("c")
        base = wid * short_elems + _L * jnp.minimum(wid, nbig)
        cp_c = pltpu.async_copy(
            conf_hbm.at[pl.ds(base, short_elems)],
            conf_v.at[pl.ds(0, short_elems)], sem)
        cp_p = pltpu.async_copy(
            pred_hbm.at[pl.ds(base, short_elems)],
            pred_v.at[pl.ds(0, short_elems)], sem)
        cp_l = pltpu.async_copy(
            lab_hbm.at[pl.ds(base, short_elems)],
            lab_v.at[pl.ds(0, short_elems)], sem)

        zero = jnp.zeros((_L,), jnp.float32)
        for r in range(_L):
            acc_v[r, :] = zero
        lane = lax.iota(jnp.int32, _L)
        tabv = lane.astype(jnp.float32) / jnp.float32(_N_BINS)

        if nbig:
            @pl.when(wid < nbig)
            def _():
                tail = base + short_elems
                pltpu.sync_copy(conf_hbm.at[pl.ds(tail, _L)],
                                conf_v.at[pl.ds(short_elems, _L)])
                pltpu.sync_copy(pred_hbm.at[pl.ds(tail, _L)],
                                pred_v.at[pl.ds(short_elems, _L)])
                pltpu.sync_copy(lab_hbm.at[pl.ds(tail, _L)],
                                lab_v.at[pl.ds(short_elems, _L)])

            @pl.when(wid >= nbig)
            def _():
                conf_v[pl.ds(short_elems, _L)] = zero

        cp_c.wait()
        cp_p.wait()
        cp_l.wait()

        def one(off):
            c = conf_v[pl.ds(off, _L)]
            p = pred_v[pl.ds(off, _L)]
            l = lab_v[pl.ds(off, _L)]
            a = jnp.where(p == l, jnp.float32(1.0), jnp.float32(0.0))
            d = c - a
            ti = (c * jnp.float32(15.0)).astype(jnp.int32)
            bi = lax.bitcast_convert_type(
                jnp.minimum(lax.bitcast_convert_type(ti, jnp.uint32),
                            jnp.uint32(14)),
                jnp.int32)
            lo = jnp.take_along_axis(tabv, bi, axis=0)
            b = bi - (c == lo).astype(jnp.int32)
            plsc.addupdate_scatter(acc_v, [lane, b], d,
                                   mask=c > jnp.float32(0.0))

        @plsc.parallel_loop(0, nv_main * _L, _L, unroll=_UNROLL)
        def _(off):
            one(off)

        for v in range(nv_main, nv):
            one(v * _L)

        tot = acc_v[0, :]
        for r in range(1, _L):
            tot = tot + acc_v[r, :]
        buf_v[...] = tot
        pltpu.sync_copy(buf_v, out_hbm.at[wid])

    mesh = plsc.VectorSubcoreMesh(
        core_axis_name="c", subcore_axis_name="s",
        num_cores=num_cores, num_subcores=num_subcores)
    kfn = pl.kernel(
        body,
        out_type=jax.ShapeDtypeStruct((nw, _L), jnp.float32),
        mesh=mesh,
        compiler_params=pltpu.CompilerParams(needs_layout_passes=False),
        scratch_types=[
            pltpu.VMEM((epw,), jnp.float32),
            pltpu.VMEM((epw,), jnp.int32),
            pltpu.VMEM((epw,), jnp.int32),
            pltpu.VMEM((_L, _L), jnp.float32),
            pltpu.VMEM((_L,), jnp.float32),
            pltpu.SemaphoreType.DMA,
        ],
    )
    return kfn(conf, pred, lab)


@jax.jit
def kernel(confidences, predictions, labels):
    n = confidences.shape[0]
    parts = _ece_partials(confidences, predictions, labels,
                          num_cores=2, num_subcores=16)
    sums = parts.sum(axis=0)
    ece = jnp.abs(sums[:_N_BINS]).sum() / jnp.float32(n)
    return ece.reshape(1)
